# baseline (speedup 1.0000x reference)
import functools

import jax
import jax.numpy as jnp
from jax import lax
from jax.experimental import pallas as pl
from jax.experimental.pallas import tpu as pltpu
from jax.experimental.pallas import tpu_sc as plsc

_F = 4096
_NW = 32
_C = 8
_L = 16


def _build_permute(n_rows):
    rows_per_w = n_rows // _NW
    nch = rows_per_w // _C
    mesh = plsc.VectorSubcoreMesh(core_axis_name="c", subcore_axis_name="s")

    @functools.partial(
        pl.kernel,
        mesh=mesh,
        out_type=jax.ShapeDtypeStruct((n_rows, _F), jnp.float32),
        compiler_params=pltpu.CompilerParams(
            needs_layout_passes=False,
        ),
        scratch_types=[
            pltpu.VMEM((_F,), jnp.int32),
            pltpu.VMEM((_C, _F), jnp.float32),
            pltpu.VMEM((_C, _F), jnp.float32),
        ],
    )
    def k(x_hbm, perm_hbm, out_hbm, perm_v, in_v, out_v):
        wid = lax.axis_index("s") * 2 + lax.axis_index("c")
        pltpu.sync_copy(perm_hbm, perm_v)
        base = wid * rows_per_w

        def chunk_body(g, carry):
            row0 = base + g * _C
            pltpu.sync_copy(x_hbm.at[pl.ds(row0, _C)], in_v)

            def col_body(j, carry2):
                pcol = perm_v[pl.ds(j * _L, _L)]
                for r in range(_C):
                    rvec = jnp.full((_L,), r, jnp.int32)
                    out_v[r, pl.ds(j * _L, _L)] = plsc.load_gather(
                        in_v, [rvec, pcol]
                    )
                return carry2

            lax.fori_loop(0, _F // _L, col_body, 0)
            pltpu.sync_copy(out_v, out_hbm.at[pl.ds(row0, _C)])
            return carry

        lax.fori_loop(0, nch, chunk_body, 0)

    return k


_PERMUTE = _build_permute(4 * 4096)


def kernel(x, perm):
    b, s, f = x.shape
    out = _PERMUTE(x.reshape(b * s, f), perm.astype(jnp.int32))
    return out.reshape(b, s, f)


# speedup vs baseline: 4.2571x; 4.2571x over previous
import functools

import jax
import jax.numpy as jnp
from jax import lax
from jax.experimental import pallas as pl
from jax.experimental.pallas import tpu as pltpu
from jax.experimental.pallas import tpu_sc as plsc

_F = 4096
_NW = 32
_C = 4
_L = 16


def _build_permute(n_rows):
    rows_per_w = n_rows // _NW
    nch = rows_per_w // _C
    mesh = plsc.VectorSubcoreMesh(core_axis_name="c", subcore_axis_name="s")

    @functools.partial(
        pl.kernel,
        mesh=mesh,
        out_type=jax.ShapeDtypeStruct((n_rows, _F), jnp.float32),
        compiler_params=pltpu.CompilerParams(
            needs_layout_passes=False,
        ),
        scratch_types=[
            pltpu.VMEM((_F,), jnp.int32),
            pltpu.VMEM((_C, _F), jnp.float32),
            pltpu.VMEM((_C, _F), jnp.float32),
            pltpu.VMEM((_C, _F), jnp.float32),
            pltpu.VMEM((_C, _F), jnp.float32),
            pltpu.SemaphoreType.DMA,
            pltpu.SemaphoreType.DMA,
            pltpu.SemaphoreType.DMA,
            pltpu.SemaphoreType.DMA,
        ],
    )
    def k(x_hbm, perm_hbm, out_hbm, perm_v, in0, in1, o0, o1, si0, si1, so0, so1):
        wid = lax.axis_index("s") * 2 + lax.axis_index("c")
        pltpu.sync_copy(perm_hbm, perm_v)
        base = wid * rows_per_w
        ins, outs = (in0, in1), (o0, o1)
        sis, sos = (si0, si1), (so0, so1)

        def in_slice(g):
            return x_hbm.at[pl.ds(base + g * _C, _C)]

        def out_slice(g):
            return out_hbm.at[pl.ds(base + g * _C, _C)]

        pltpu.async_copy(in_slice(0), in0, si0)
        pltpu.async_copy(in_slice(1), in1, si1)

        @pl.loop(0, nch, step=2)
        def pair(g):
            for b in range(2):
                gb = g + b
                in_v, out_v = ins[b], outs[b]
                pltpu.make_async_copy(in_slice(gb), in_v, sis[b]).wait()

                @pl.when(gb >= 2)
                def _wait_out():
                    pltpu.make_async_copy(out_v, out_slice(gb), sos[b]).wait()

                @plsc.parallel_loop(0, _F // _L, unroll=8)
                def col(j):
                    pcol = perm_v[pl.ds(j * _L, _L)]
                    for r in range(_C):
                        rvec = jnp.full((_L,), r, jnp.int32)
                        out_v[r, pl.ds(j * _L, _L)] = plsc.load_gather(
                            in_v, [rvec, pcol]
                        )

                pltpu.async_copy(out_v, out_slice(gb), sos[b])

                @pl.when(gb + 2 < nch)
                def _next_in():
                    pltpu.async_copy(in_slice(gb + 2), in_v, sis[b])

        pltpu.make_async_copy(o0, out_slice(0), so0).wait()
        pltpu.make_async_copy(o1, out_slice(1), so1).wait()

    return k


_PERMUTE = _build_permute(4 * 4096)


def kernel(x, perm):
    b, s, f = x.shape
    out = _PERMUTE(x.reshape(b * s, f), perm.astype(jnp.int32))
    return out.reshape(b, s, f)
